# baseline (speedup 1.0000x reference)
import jax
import jax.numpy as jnp
from jax import lax
from jax.experimental import pallas as pl
from jax.experimental.pallas import tpu as pltpu
from jax.experimental.pallas import tpu_sc as plsc

_NF = 26
_NP = 325
_DT = 64
_BT = 1024
_NC = 2
_NS = 16
_RW = 16
_CW = 128
_L = 16
_NSL = _CW // _L
_G = 13
_NB = _NP // _G

_PAIRS = [(i, j) for i in range(_NF) for j in range(i + 1, _NF)]


def _sc_body(x_hbm, out_hbm, xv, ob0, ob1, sem):
    wid = lax.axis_index("s") * _NC + lax.axis_index("c")
    r0 = pl.multiple_of(lax.shift_right_logical(wid, 3) * _RW, _RW)
    c0 = pl.multiple_of((wid & 7) * _CW, _CW)
    pltpu.sync_copy(x_hbm.at[:, pl.ds(r0, _RW), pl.ds(c0, _CW)], xv)

    bufs = (ob0, ob1)

    def dst(b):
        return out_hbm.at[pl.ds(b * _G, _G), pl.ds(r0, _RW), pl.ds(c0, _CW)]

    for b in range(_NB):
        buf = bufs[b % 2]
        if b >= 2:
            pltpu.make_async_copy(buf, dst(b - 2), sem.at[b % 2]).wait()

        block_pairs = _PAIRS[b * _G:(b + 1) * _G]

        def r_step(v, acc, buf=buf, block_pairs=block_pairs):
            r = lax.shift_right_logical(v, 1)
            ho = (v & 1) * (_CW // 2)
            for g, (i, j) in enumerate(block_pairs):
                for c in range(_NSL // 2):
                    sl = pl.ds(ho + c * _L, _L)
                    buf[g, r, sl] = xv[i, r, sl] * xv[j, r, sl]
            return acc

        lax.fori_loop(0, 2 * _RW, r_step, 0)

        pltpu.async_copy(buf, dst(b), sem.at[b % 2])

    for b in (_NB - 2, _NB - 1):
        pltpu.make_async_copy(bufs[b % 2], dst(b), sem.at[b % 2]).wait()


def kernel(x):
    xt = jnp.transpose(x, (0, 2, 1))
    k = pl.kernel(
        _sc_body,
        out_type=jax.ShapeDtypeStruct((_NP, _DT, _BT), jnp.float32),
        mesh=plsc.VectorSubcoreMesh(core_axis_name="c", subcore_axis_name="s"),
        scratch_types=[
            pltpu.VMEM((_NF, _RW, _CW), jnp.float32),
            pltpu.VMEM((_G, _RW, _CW), jnp.float32),
            pltpu.VMEM((_G, _RW, _CW), jnp.float32),
            pltpu.SemaphoreType.DMA((2,)),
        ],
    )
    out_t = k(xt)
    return jnp.transpose(out_t, (0, 2, 1))


# speedup vs baseline: 1.0530x; 1.0530x over previous
import jax
import jax.numpy as jnp
from jax import lax
from jax.experimental import pallas as pl
from jax.experimental.pallas import tpu as pltpu
from jax.experimental.pallas import tpu_sc as plsc

_NF = 26
_NP = 325
_DT = 64
_BT = 1024
_NC = 2
_NS = 16
_RW = 16
_CW = 128
_L = 16
_NSL = _CW // _L
_G = 13
_NB = _NP // _G


def _sc_body(x_hbm, out_hbm, xv, ob, sem):
    wid = lax.axis_index("s") * _NC + lax.axis_index("c")
    r0 = pl.multiple_of(lax.shift_right_logical(wid, 3) * _RW, _RW)
    c0 = pl.multiple_of((wid & 7) * _CW, _CW)
    pltpu.sync_copy(x_hbm.at[:, pl.ds(r0, _RW), pl.ds(c0, _CW)], xv)

    def dst(p0):
        return out_hbm.at[pl.ds(p0, _G), pl.ds(r0, _RW), pl.ds(c0, _CW)]

    def blk_step(b, carry):
        ii, jj = carry
        slot = b % 2
        p0 = b * _G

        @pl.when(b >= 2)
        def _reclaim():
            pltpu.make_async_copy(
                ob.at[slot], dst(p0 - 2 * _G), sem.at[slot]
            ).wait()

        iis, jjs = [], []
        ci, cj = ii, jj
        for _ in range(_G):
            iis.append(ci)
            jjs.append(cj)
            cn = cj + 1
            wrap = cn == _NF
            ci = jnp.where(wrap, ci + 1, ci)
            cj = jnp.where(wrap, ci + 1, cn)

        def r_step(r, acc):
            for g in range(_G):
                for c in range(_NSL):
                    sl = pl.ds(c * _L, _L)
                    ob[slot, g, r, sl] = xv[iis[g], r, sl] * xv[jjs[g], r, sl]
            return acc

        lax.fori_loop(0, _RW, r_step, 0)

        pltpu.async_copy(ob.at[slot], dst(p0), sem.at[slot])
        return ci, cj

    lax.fori_loop(0, _NB, blk_step, (jnp.int32(0), jnp.int32(1)))

    for b in (_NB - 2, _NB - 1):
        pltpu.make_async_copy(ob.at[b % 2], dst(b * _G), sem.at[b % 2]).wait()


def kernel(x):
    xt = jnp.transpose(x, (0, 2, 1))
    k = pl.kernel(
        _sc_body,
        out_type=jax.ShapeDtypeStruct((_NP, _DT, _BT), jnp.float32),
        mesh=plsc.VectorSubcoreMesh(core_axis_name="c", subcore_axis_name="s"),
        scratch_types=[
            pltpu.VMEM((_NF, _RW, _CW), jnp.float32),
            pltpu.VMEM((2, _G, _RW, _CW), jnp.float32),
            pltpu.SemaphoreType.DMA((2,)),
        ],
    )
    out_t = k(xt)
    return jnp.transpose(out_t, (0, 2, 1))


# speedup vs baseline: 2.6877x; 2.5525x over previous
import jax
import jax.numpy as jnp
from jax import lax
from jax.experimental import pallas as pl
from jax.experimental.pallas import tpu as pltpu
from jax.experimental.pallas import tpu_sc as plsc

_NF = 26
_NP = 325
_DT = 64
_BT = 1024
_NC = 2
_NS = 16
_RW = 16
_CW = 128
_L = 16
_NSL = _CW // _L
_G = 13
_NB = _NP // _G


def _advance(ci, cj):
    cn = cj + 1
    wrap = cn == _NF
    ci = jnp.where(wrap, ci + 1, ci)
    cj = jnp.where(wrap, ci + 1, cn)
    return ci, cj


def _sc_body(x_hbm, out_hbm, xv, ob0, ob1, sem):
    wid = lax.axis_index("s") * _NC + lax.axis_index("c")
    r0 = pl.multiple_of(lax.shift_right_logical(wid, 3) * _RW, _RW)
    c0 = pl.multiple_of((wid & 7) * _CW, _CW)
    pltpu.sync_copy(x_hbm.at[:, pl.ds(r0, _RW), pl.ds(c0, _CW)], xv)

    def dst(p0):
        return out_hbm.at[pl.ds(p0, _G), pl.ds(r0, _RW), pl.ds(c0, _CW)]

    def run_block(buf, sl_sem, p0, ci, cj):
        iis, jjs = [], []
        for _ in range(_G):
            iis.append(ci)
            jjs.append(cj)
            ci, cj = _advance(ci, cj)

        def r_step(r, acc):
            for c in range(_NSL):
                sl = pl.ds(c * _L, _L)
                avs = [xv[iis[g], r, sl] for g in range(_G)]
                bvs = [xv[jjs[g], r, sl] for g in range(_G)]
                for g in range(_G):
                    buf[g, r, sl] = avs[g] * bvs[g]
            return acc

        lax.fori_loop(0, _RW, r_step, 0)
        pltpu.async_copy(buf, dst(p0), sl_sem)
        return ci, cj

    def super_step(s, carry):
        ci, cj = carry
        p0 = s * 2 * _G

        @pl.when(s >= 1)
        def _reclaim0():
            pltpu.make_async_copy(ob0, dst(p0 - 2 * _G), sem.at[0]).wait()

        ci, cj = run_block(ob0, sem.at[0], p0, ci, cj)

        @pl.when(s >= 1)
        def _reclaim1():
            pltpu.make_async_copy(ob1, dst(p0 - _G), sem.at[1]).wait()

        ci, cj = run_block(ob1, sem.at[1], p0 + _G, ci, cj)
        return ci, cj

    ci, cj = lax.fori_loop(
        0, (_NB - 1) // 2, super_step, (jnp.int32(0), jnp.int32(1))
    )

    pltpu.make_async_copy(ob0, dst((_NB - 3) * _G), sem.at[0]).wait()
    run_block(ob0, sem.at[0], (_NB - 1) * _G, ci, cj)

    pltpu.make_async_copy(ob1, dst((_NB - 2) * _G), sem.at[1]).wait()
    pltpu.make_async_copy(ob0, dst((_NB - 1) * _G), sem.at[0]).wait()


def kernel(x):
    xt = jnp.transpose(x, (0, 2, 1))
    k = pl.kernel(
        _sc_body,
        out_type=jax.ShapeDtypeStruct((_NP, _DT, _BT), jnp.float32),
        mesh=plsc.VectorSubcoreMesh(core_axis_name="c", subcore_axis_name="s"),
        scratch_types=[
            pltpu.VMEM((_NF, _RW, _CW), jnp.float32),
            pltpu.VMEM((_G, _RW, _CW), jnp.float32),
            pltpu.VMEM((_G, _RW, _CW), jnp.float32),
            pltpu.SemaphoreType.DMA((2,)),
        ],
    )
    out_t = k(xt)
    return jnp.transpose(out_t, (0, 2, 1))
